# Initial kernel scaffold; baseline (speedup 1.0000x reference)
import jax
import jax.numpy as jnp
from jax.experimental import pallas as pl


def kernel(edge_index, embedding):
    raise NotImplementedError("write your pallas kernel here")


# speedup vs baseline: 6.1556x; 6.1556x over previous
import functools

import jax
import jax.numpy as jnp
from jax import lax
from jax.experimental import pallas as pl
from jax.experimental.pallas import tpu as pltpu
from jax.experimental.pallas import tpu_sc as plsc

N_NODES = 50000
D = 64
E = 800000

NPAD = 50176
HALF = NPAD // 2
ACC = HALF + 128
GARB = HALF
C = 128
NTILE = 16
EP = 16 * C * ((E + 16 * C - 1) // (16 * C))
PT = EP // NTILE
G = PT // C

_MESH = plsc.VectorSubcoreMesh(
    core_axis_name="c", subcore_axis_name="s", num_cores=2, num_subcores=16)
_SC_PARAMS = pltpu.CompilerParams(use_tc_tiling_on_sc=False)


def _local_idx(col_v, idx_v, sc_lo):
    for i in range(C // 16):
        v = col_v[pl.ds(i * 16, 16)] - sc_lo
        oob = (v < 0) | (v >= HALF)
        idx_v[pl.ds(i * 16, 16)] = jnp.where(oob, GARB, v)


@functools.partial(
    pl.kernel,
    out_type=jax.ShapeDtypeStruct((NPAD, D), jnp.float32),
    mesh=_MESH,
    scratch_types=[
        pltpu.VMEM((C,), jnp.int32),
        pltpu.VMEM((C,), jnp.int32),
        pltpu.VMEM((C,), jnp.int32),
        pltpu.VMEM((C, D), jnp.float32),
        pltpu.VMEM_SHARED((ACC, D), jnp.float32),
        pltpu.SemaphoreType.DMA,
    ],
    compiler_params=_SC_PARAMS,
)
def _layer_sc(s_hbm, row_hbm, col_hbm, z_hbm, out_hbm,
              row_v, col_v, idx_v, msg_v, acc_sh, sem):
    cid = lax.axis_index("c")
    sid = lax.axis_index("s")
    sc_lo = cid * HALF
    zrows = ACC // NTILE
    zr = sid * zrows
    pltpu.sync_copy(z_hbm.at[pl.ds(zr, zrows)], acc_sh.at[pl.ds(zr, zrows)])
    plsc.subcore_barrier()

    ebase = sid * PT

    def chunk(g, carry):
        base = ebase + g * C
        pltpu.sync_copy(row_hbm.at[pl.ds(base, C)], row_v)
        pltpu.sync_copy(col_hbm.at[pl.ds(base, C)], col_v)
        pltpu.async_copy(s_hbm.at[row_v], msg_v, sem).wait()
        _local_idx(col_v, idx_v, sc_lo)
        pltpu.sync_copy(msg_v, acc_sh.at[idx_v], add=True)
        return carry

    lax.fori_loop(0, G, chunk, 0)
    plsc.subcore_barrier()
    wrows = HALF // NTILE
    wr = sid * wrows
    pltpu.sync_copy(acc_sh.at[pl.ds(wr, wrows)],
                    out_hbm.at[pl.ds(sc_lo + wr, wrows)])


@functools.partial(
    pl.kernel,
    out_type=jax.ShapeDtypeStruct((NPAD, 16), jnp.float32),
    mesh=_MESH,
    scratch_types=[
        pltpu.VMEM((C,), jnp.int32),
        pltpu.VMEM((C,), jnp.int32),
        pltpu.VMEM((C, 16), jnp.float32),
        pltpu.VMEM_SHARED((ACC, 16), jnp.float32),
    ],
    compiler_params=_SC_PARAMS,
)
def _deg_sc(col_hbm, z16_hbm, ones_hbm, out_hbm, col_v, idx_v, ones_v, acc_sh):
    cid = lax.axis_index("c")
    sid = lax.axis_index("s")
    sc_lo = cid * HALF
    zrows = ACC // NTILE
    zr = sid * zrows
    pltpu.sync_copy(z16_hbm.at[pl.ds(zr, zrows)], acc_sh.at[pl.ds(zr, zrows)])
    pltpu.sync_copy(ones_hbm, ones_v)
    plsc.subcore_barrier()

    ebase = sid * PT

    def chunk(g, carry):
        base = ebase + g * C
        pltpu.sync_copy(col_hbm.at[pl.ds(base, C)], col_v)
        _local_idx(col_v, idx_v, sc_lo)
        pltpu.sync_copy(ones_v, acc_sh.at[idx_v], add=True)
        return carry

    lax.fori_loop(0, G, chunk, 0)
    plsc.subcore_barrier()
    wrows = HALF // NTILE
    wr = sid * wrows
    pltpu.sync_copy(acc_sh.at[pl.ds(wr, wrows)],
                    out_hbm.at[pl.ds(sc_lo + wr, wrows)])


_R = 512
_GRID = NPAD // _R


def _norm_body(deg_ref, emb_ref, s_ref, dis_ref):
    deg = deg_ref[:, 0:1]
    dis = jnp.where(deg > 0.0, lax.rsqrt(deg), 0.0)
    dis64 = jnp.broadcast_to(dis, (_R, D))
    s_ref[...] = dis64 * emb_ref[...]
    dis_ref[...] = dis64


_norm_tc = pl.pallas_call(
    _norm_body,
    grid=(_GRID,),
    in_specs=[pl.BlockSpec((_R, 16), lambda i: (i, 0)),
              pl.BlockSpec((_R, D), lambda i: (i, 0))],
    out_specs=[pl.BlockSpec((_R, D), lambda i: (i, 0)),
               pl.BlockSpec((_R, D), lambda i: (i, 0))],
    out_shape=[jax.ShapeDtypeStruct((NPAD, D), jnp.float32),
               jax.ShapeDtypeStruct((NPAD, D), jnp.float32)],
)


def _scale_body(acc_ref, dis_ref, sumin_ref, s_ref, sumout_ref):
    da = dis_ref[...] * acc_ref[...]
    s_ref[...] = dis_ref[...] * da
    sumout_ref[...] = sumin_ref[...] + da


_scale_tc = pl.pallas_call(
    _scale_body,
    grid=(_GRID,),
    in_specs=[pl.BlockSpec((_R, D), lambda i: (i, 0))] * 3,
    out_specs=[pl.BlockSpec((_R, D), lambda i: (i, 0))] * 2,
    out_shape=[jax.ShapeDtypeStruct((NPAD, D), jnp.float32),
               jax.ShapeDtypeStruct((NPAD, D), jnp.float32)],
)


def _final_body(acc_ref, dis_ref, sumin_ref, out_ref):
    out_ref[...] = (sumin_ref[...] + dis_ref[...] * acc_ref[...]) * 0.25


_final_tc = pl.pallas_call(
    _final_body,
    grid=(_GRID,),
    in_specs=[pl.BlockSpec((_R, D), lambda i: (i, 0))] * 3,
    out_specs=pl.BlockSpec((_R, D), lambda i: (i, 0)),
    out_shape=jax.ShapeDtypeStruct((NPAD, D), jnp.float32),
)


def kernel(edge_index, embedding):
    row = edge_index[0].astype(jnp.int32)
    col = edge_index[1].astype(jnp.int32)
    emb = jnp.pad(embedding, ((0, NPAD - N_NODES), (0, 0)))
    pad_e = EP - E
    row_p = jnp.concatenate([row, jnp.zeros((pad_e,), jnp.int32)])
    col_p = jnp.concatenate([col, jnp.full((pad_e,), NPAD, jnp.int32)])
    z64 = jnp.zeros((ACC, D), jnp.float32)
    z16 = jnp.zeros((ACC, 16), jnp.float32)
    ones16 = jnp.ones((C, 16), jnp.float32)

    degt = _deg_sc(col_p, z16, ones16)
    s0, dis64 = _norm_tc(degt, emb)
    acc1 = _layer_sc(s0, row_p, col_p, z64)
    s1, summ = _scale_tc(acc1, dis64, emb)
    acc2 = _layer_sc(s1, row_p, col_p, z64)
    s2, summ = _scale_tc(acc2, dis64, summ)
    acc3 = _layer_sc(s2, row_p, col_p, z64)
    out = _final_tc(acc3, dis64, summ)
    return out[:N_NODES]
